# Initial kernel scaffold; baseline (speedup 1.0000x reference)
import jax
import jax.numpy as jnp
from jax.experimental import pallas as pl


def kernel(h, r, fc_w, attn_w, edge_index):
    raise NotImplementedError("write your pallas kernel here")


# speedup vs baseline: 5.3833x; 5.3833x over previous
import functools

import jax
import jax.numpy as jnp
from jax import lax
from jax.experimental import pallas as pl
from jax.experimental.pallas import tpu as pltpu
from jax.experimental.pallas import tpu_sc as plsc

N = 10000
E = 320000
D = 128
NC = 2
NS = 16
NW = NC * NS
L = 16

EPW = E // NW
C = 80
NCHUNK = EPW // C
NG = C // L
NODE_W = 640
N_PAD = NS * NODE_W

def _mesh():
    return plsc.VectorSubcoreMesh(
        core_axis_name="c", subcore_axis_name="s", num_cores=NC,
        num_subcores=NS
    )


def _hr_body(h_hbm, r_hbm, dst_hbm,
             hr_out,
             dst2_v, hb, rb,
             sem_h0, sem_h1, sem_r0, sem_r1, sem_w0, sem_w1):
    c = lax.axis_index("c")
    s = lax.axis_index("s")
    wid = s * NC + c
    lanes = lax.iota(jnp.int32, L)
    sem_h = (sem_h0, sem_h1)
    sem_r = (sem_r0, sem_r1)
    sem_w = (sem_w0, sem_w1)

    pltpu.sync_copy(dst_hbm.at[wid], dst2_v)

    def issue(g, b):
        pltpu.async_copy(h_hbm.at[dst2_v.at[g]], hb.at[b], sem_h[b])
        pltpu.async_copy(r_hbm.at[pl.ds(wid * EPW + g * C, C), :], rb.at[b],
                         sem_r[b])

    issue(0, 0)

    def compute(g, b):
        @pl.when(g >= 1)
        def _():
            pltpu.make_async_copy(
                hb.at[1 - b],
                hr_out.at[pl.ds(wid * EPW + (g - 1) * C, C), :],
                sem_w[1 - b]).wait()

        @pl.when(g + 1 < NCHUNK)
        def _():
            issue(g + 1, 1 - b)

        pltpu.make_async_copy(h_hbm.at[dst2_v.at[g]], hb.at[b],
                              sem_h[b]).wait()
        pltpu.make_async_copy(r_hbm.at[pl.ds(wid * EPW + g * C, C), :],
                              rb.at[b], sem_r[b]).wait()

        hrr = hb.at[b]
        rr = rb.at[b]

        def mul_body(i, _):
            rv = jnp.full((L,), i // (D // L), jnp.int32)
            cv = jnp.full((L,), (i % (D // L)) * L, jnp.int32) + lanes
            hv = plsc.load_gather(hrr, [rv, cv])
            rv2 = plsc.load_gather(rr, [rv, cv])
            plsc.store_scatter(hrr, [rv, cv], hv * rv2)
            return 0
        lax.fori_loop(0, C * D // L, mul_body, 0)

        pltpu.async_copy(hb.at[b],
                         hr_out.at[pl.ds(wid * EPW + g * C, C), :],
                         sem_w[b])

    def iter_body(g, _):
        @pl.when(g % 2 == 0)
        def _():
            compute(g, 0)

        @pl.when(g % 2 == 1)
        def _():
            compute(g, 1)
        return 0
    lax.fori_loop(0, NCHUNK, iter_body, 0)
    pltpu.make_async_copy(
        hb.at[(NCHUNK - 1) % 2],
        hr_out.at[pl.ds(wid * EPW + (NCHUNK - 1) * C, C), :],
        sem_w[(NCHUNK - 1) % 2]).wait()


@functools.cache
def _launch_hr():
    return pl.kernel(
        _hr_body,
        out_type=[jax.ShapeDtypeStruct((E, D), jnp.float32)],
        mesh=_mesh(),
        compiler_params=pltpu.CompilerParams(needs_layout_passes=False),
        scratch_types=[
            pltpu.VMEM((NCHUNK, C), jnp.int32),
            pltpu.VMEM((2, C, D), jnp.float32),
            pltpu.VMEM((2, C, D), jnp.float32),
            pltpu.SemaphoreType.DMA,
            pltpu.SemaphoreType.DMA,
            pltpu.SemaphoreType.DMA,
            pltpu.SemaphoreType.DMA,
            pltpu.SemaphoreType.DMA,
            pltpu.SemaphoreType.DMA,
        ],
    )


def _ps_body(h_ref, fc_ref, attn_ref, ps_ref):
    h2 = jnp.dot(h_ref[...], fc_ref[...].T, preferred_element_type=jnp.float32)
    h2 = h2.astype(jnp.bfloat16).astype(jnp.float32)
    a1 = attn_ref[:, :D].astype(jnp.bfloat16).astype(jnp.float32)
    ps_ref[...] = jnp.sum(h2 * a1, axis=1, keepdims=True)


def _precompute_ps(h, fc_w, attn_w):
    return pl.pallas_call(
        _ps_body,
        out_shape=jax.ShapeDtypeStruct((N, 1), jnp.float32),
    )(h, fc_w, attn_w)


EBLK = 4000


def _q_body(hr_ref, fc_ref, attn_ref, q_ref):
    dr = jnp.dot(hr_ref[...], fc_ref[...].T,
                 preferred_element_type=jnp.float32)
    dr = dr.astype(jnp.bfloat16).astype(jnp.float32)
    a2 = attn_ref[:, D:].astype(jnp.bfloat16).astype(jnp.float32)
    q_ref[...] = jnp.sum(dr * a2, axis=1, keepdims=True)


def _precompute_q(hr, fc_w, attn_w):
    return pl.pallas_call(
        _q_body,
        grid=(E // EBLK,),
        in_specs=[
            pl.BlockSpec((EBLK, D), lambda i: (i, 0)),
            pl.BlockSpec((D, D), lambda i: (0, 0)),
            pl.BlockSpec((1, 2 * D), lambda i: (0, 0)),
        ],
        out_specs=pl.BlockSpec((EBLK, 1), lambda i: (i, 0)),
        out_shape=jax.ShapeDtypeStruct((E, 1), jnp.float32),
    )(hr, fc_w, attn_w)


def _a_body(ps_hbm, q_hbm, src_hbm, dst_hbm,
            e_out, m_out, m_stage,
            ps_v, m_v, src2_v, dst2_v, e_loc, q_loc, acc_v, tmp_v):
    c = lax.axis_index("c")
    s = lax.axis_index("s")
    wid = s * NC + c
    lanes = lax.iota(jnp.int32, L)

    pltpu.sync_copy(ps_hbm, ps_v)
    pltpu.sync_copy(src_hbm.at[wid], src2_v)
    pltpu.sync_copy(dst_hbm.at[wid], dst2_v)
    pltpu.sync_copy(q_hbm.at[pl.ds(wid * EPW, EPW)], q_loc)

    neg_inf = jnp.full((L,), -jnp.inf, jnp.float32)

    def _minit(i, _):
        plsc.store_scatter(m_v, [jnp.full((L,), i * L, jnp.int32) + lanes],
                           neg_inf)
        return 0
    lax.fori_loop(0, N_PAD // L, _minit, 0)

    def grp(i, _):
        iv = jnp.full((L,), i * L, jnp.int32) + lanes
        g = i // NG
        col = (i % NG) * L + lanes
        gv = jnp.full((L,), g, jnp.int32)
        srcv = plsc.load_gather(src2_v, [gv, col])
        dstv = plsc.load_gather(dst2_v, [gv, col])
        pv = plsc.load_gather(ps_v, [srcv])
        qv = plsc.load_gather(q_loc, [iv])
        e = pv + qv
        e = jnp.where(e >= 0.0, e, 0.01 * e) / jnp.float32(0.05)
        plsc.store_scatter(e_loc, [iv], e)

        def mbody(_):
            cur = plsc.load_gather(m_v, [dstv])
            plsc.store_scatter(m_v, [dstv], e, mask=e > cur)
            cur2 = plsc.load_gather(m_v, [dstv])
            return jnp.any(e > cur2)
        lax.while_loop(lambda cont: cont, mbody, jnp.bool_(True))
        return 0
    lax.fori_loop(0, EPW // L, grp, 0)

    pltpu.sync_copy(e_loc, e_out.at[pl.ds(wid * EPW, EPW)])

    pltpu.sync_copy(m_v, m_stage.at[c, s])
    plsc.subcore_barrier()
    pltpu.sync_copy(m_stage.at[c, 0, pl.ds(s * NODE_W, NODE_W)], acc_v)

    def tmerge(t, _):
        pltpu.sync_copy(m_stage.at[c, t, pl.ds(s * NODE_W, NODE_W)], tmp_v)

        def vm(i, _):
            iv = jnp.full((L,), i * L, jnp.int32) + lanes
            a = plsc.load_gather(acc_v, [iv])
            bb = plsc.load_gather(tmp_v, [iv])
            plsc.store_scatter(acc_v, [iv], jnp.maximum(a, bb))
            return 0
        lax.fori_loop(0, NODE_W // L, vm, 0)
        return 0
    lax.fori_loop(1, NS, tmerge, 0)
    pltpu.sync_copy(acc_v, m_out.at[c, pl.ds(s * NODE_W, NODE_W)])


@functools.cache
def _launch_a():
    return pl.kernel(
        _a_body,
        out_type=[
            jax.ShapeDtypeStruct((E,), jnp.float32),
            jax.ShapeDtypeStruct((NC, N_PAD), jnp.float32),
            jax.ShapeDtypeStruct((NC, NS, N_PAD), jnp.float32),
        ],
        mesh=_mesh(),
        compiler_params=pltpu.CompilerParams(needs_layout_passes=False),
        scratch_types=[
            pltpu.VMEM((N,), jnp.float32),
            pltpu.VMEM((N_PAD,), jnp.float32),
            pltpu.VMEM((NCHUNK, C), jnp.int32),
            pltpu.VMEM((NCHUNK, C), jnp.int32),
            pltpu.VMEM((EPW,), jnp.float32),
            pltpu.VMEM((EPW,), jnp.float32),
            pltpu.VMEM((NODE_W,), jnp.float32),
            pltpu.VMEM((NODE_W,), jnp.float32),
        ],
    )


NQ = 4
HD = D // NQ
EPT_B = E // NS
NCHUNK_B = EPT_B // C


def _b_body(hsplit_hbm, e_hbm, m_hbm, src_hbm, dst_hbm,
            den_out, out_full, den_stage,
            m_v, den_v, claim_v, src2_v, dst2_v, e_loc, hb,
            acc_v, tmp_v, out_sh,
            sem_g0, sem_g1, sem_s0, sem_s1):
    c = lax.axis_index("c")
    s = lax.axis_index("s")
    lanes = lax.iota(jnp.int32, L)
    sem_g = (sem_g0, sem_g1)
    sem_s = (sem_s0, sem_s1)

    pltpu.sync_copy(src_hbm.at[s], src2_v)
    pltpu.sync_copy(dst_hbm.at[s], dst2_v)
    pltpu.sync_copy(e_hbm.at[pl.ds(s * EPT_B, EPT_B)], e_loc)
    pltpu.sync_copy(m_hbm.at[0], m_v)
    pltpu.sync_copy(m_hbm.at[1], den_v)

    def _mmerge(i, _):
        iv = jnp.full((L,), i * L, jnp.int32) + lanes
        a = plsc.load_gather(m_v, [iv])
        bb = plsc.load_gather(den_v, [iv])
        plsc.store_scatter(m_v, [iv], jnp.maximum(a, bb))
        plsc.store_scatter(den_v, [iv], jnp.zeros((L,), jnp.float32))
        return 0
    lax.fori_loop(0, N_PAD // L, _mmerge, 0)

    zero = jnp.zeros((L,), jnp.float32)

    def _zero_window():
        def _hz(i, _):
            rv = jnp.full((L,), i // (HD // L), jnp.int32)
            cv = jnp.full((L,), (i % (HD // L)) * L, jnp.int32) + lanes
            plsc.store_scatter(hb.at[0], [rv, cv], zero)
            return 0
        lax.fori_loop(0, C * HD // L, _hz, 0)
        for qq in range(NODE_W // C):
            pltpu.sync_copy(hb.at[0],
                            out_sh.at[pl.ds(s * NODE_W + qq * C, C), :])

    def _run_pass(q, h_q):
        def issue(g, b):
            pltpu.async_copy(h_q.at[src2_v.at[g]], hb.at[b], sem_g[b])

        issue(0, 0)

        def compute(g, b):
            @pl.when(g + 1 < NCHUNK_B)
            def _():
                issue(g + 1, 1 - b)

            pltpu.make_async_copy(h_q.at[src2_v.at[g]], hb.at[b],
                                  sem_g[b]).wait()

            hr = hb.at[b]
            gv = jnp.full((L,), g, jnp.int32)
            for t in range(NG):
                colv = t * L + lanes
                eidx = jnp.full((L,), g * C + t * L, jnp.int32) + lanes
                if q == 0:
                    dstv = plsc.load_gather(dst2_v, [gv, colv])
                    ev = plsc.load_gather(e_loc, [eidx])
                    mv = plsc.load_gather(m_v, [dstv])
                    ex = jnp.exp(ev - mv)
                    plsc.store_scatter(e_loc, [eidx], ex)

                    @pl.when(c == 0)
                    def _():
                        def dbody(active):
                            plsc.store_scatter(claim_v, [dstv], lanes,
                                               mask=active)
                            winner = plsc.load_gather(claim_v, [dstv])
                            win = active & (winner == lanes)
                            cur = plsc.load_gather(den_v, [dstv])
                            plsc.store_scatter(den_v, [dstv], cur + ex,
                                               mask=win)
                            return active & jnp.logical_not(win)
                        lax.while_loop(lambda a: jnp.any(a), dbody,
                                       jnp.ones((L,), jnp.bool_))
                else:
                    ex = plsc.load_gather(e_loc, [eidx])

                for j in range(L):
                    bc = jnp.full(
                        (L,),
                        jnp.sum(jnp.where(lanes == j, ex, jnp.float32(0.0))))
                    rowv = jnp.full((L,), t * L + j, jnp.int32)
                    for k in range(HD // L):
                        cv = k * L + lanes
                        hv = plsc.load_gather(hr, [rowv, cv])
                        plsc.store_scatter(hr, [rowv, cv], hv * bc)

            pltpu.sync_copy(hb.at[b], out_sh.at[dst2_v.at[g]], add=True)

        def iter_body(g, _):
            @pl.when(g % 2 == 0)
            def _():
                compute(g, 0)

            @pl.when(g % 2 == 1)
            def _():
                compute(g, 1)
            return 0
        lax.fori_loop(0, NCHUNK_B, iter_body, 0)

        plsc.subcore_barrier()

    for q in range(NQ // NC):
        qid = c * (NQ // NC) + q
        _zero_window()
        plsc.subcore_barrier()
        _run_pass(q, hsplit_hbm.at[qid])
        pltpu.sync_copy(out_sh.at[pl.ds(s * NODE_W, NODE_W), :],
                        out_full.at[qid, pl.ds(s * NODE_W, NODE_W), :])
        plsc.subcore_barrier()

    @pl.when(c == 0)
    def _():
        pltpu.sync_copy(den_v, den_stage.at[s])
    plsc.subcore_barrier()

    @pl.when(c == 0)
    def _():
        pltpu.sync_copy(den_stage.at[0, pl.ds(s * NODE_W, NODE_W)], acc_v)

        def tmerge(t, _):
            pltpu.sync_copy(den_stage.at[t, pl.ds(s * NODE_W, NODE_W)],
                            tmp_v)

            def vs(i, _):
                iv = jnp.full((L,), i * L, jnp.int32) + lanes
                a = plsc.load_gather(acc_v, [iv])
                bb = plsc.load_gather(tmp_v, [iv])
                plsc.store_scatter(acc_v, [iv], a + bb)
                return 0
            lax.fori_loop(0, NODE_W // L, vs, 0)
            return 0
        lax.fori_loop(1, NS, tmerge, 0)
        pltpu.sync_copy(acc_v, den_out.at[pl.ds(s * NODE_W, NODE_W)])


@functools.cache
def _launch_b():
    return pl.kernel(
        _b_body,
        out_type=[
            jax.ShapeDtypeStruct((N_PAD,), jnp.float32),
            jax.ShapeDtypeStruct((NQ, N_PAD, HD), jnp.float32),
            jax.ShapeDtypeStruct((NS, N_PAD), jnp.float32),
        ],
        mesh=_mesh(),
        compiler_params=pltpu.CompilerParams(
            needs_layout_passes=False, use_tc_tiling_on_sc=False),
        scratch_types=[
            pltpu.VMEM((N_PAD,), jnp.float32),
            pltpu.VMEM((N_PAD,), jnp.float32),
            pltpu.VMEM((N_PAD,), jnp.int32),
            pltpu.VMEM((NCHUNK_B, C), jnp.int32),
            pltpu.VMEM((NCHUNK_B, C), jnp.int32),
            pltpu.VMEM((EPT_B,), jnp.float32),
            pltpu.VMEM((2, C, HD), jnp.float32),
            pltpu.VMEM((NODE_W,), jnp.float32),
            pltpu.VMEM((NODE_W,), jnp.float32),
            pltpu.VMEM_SHARED((N_PAD, HD), jnp.float32),
            pltpu.SemaphoreType.DMA,
            pltpu.SemaphoreType.DMA,
            pltpu.SemaphoreType.DMA,
            pltpu.SemaphoreType.DMA,
        ],
    )


def _merge_body(of_ref, dp_ref, out_ref):
    den = dp_ref[...]
    den = jnp.where(den > 0.0, den, 1.0)
    out_ref[...] = jnp.concatenate(
        [of_ref[0], of_ref[1], of_ref[2], of_ref[3]], axis=1) / den


def _merge(out_full, den):
    return pl.pallas_call(
        _merge_body,
        out_shape=jax.ShapeDtypeStruct((N_PAD, D), jnp.float32),
    )(out_full, den)


def kernel(h, r, fc_w, attn_w, edge_index):
    src = edge_index[0].astype(jnp.int32).reshape(NW, NCHUNK, C)
    dst = edge_index[1].astype(jnp.int32).reshape(NW, NCHUNK, C)
    hr = _launch_hr()(h, r, dst)[0]
    ps = _precompute_ps(h, fc_w, attn_w).reshape(N)
    q = _precompute_q(hr, fc_w, attn_w).reshape(E)
    e_all, m_part, _ = _launch_a()(ps, q, src, dst)
    srcb = edge_index[0].astype(jnp.int32).reshape(NS, NCHUNK_B, C)
    dstb = edge_index[1].astype(jnp.int32).reshape(NS, NCHUNK_B, C)
    hsplit = jnp.stack([h[:, i * HD:(i + 1) * HD] for i in range(NQ)])
    den, out_full, _ = _launch_b()(hsplit, e_all, m_part, srcb, dstb)
    out = _merge(out_full, den.reshape(N_PAD, 1))
    return out[:N]
